# baseline (speedup 1.0000x reference)
import functools

import jax
import jax.numpy as jnp
from jax import lax
from jax.experimental import pallas as pl
from jax.experimental.pallas import tpu as pltpu
from jax.experimental.pallas import tpu_sc as plsc

D = 128
H = 64
CHUNK = 128
NW = 32


def _absdiff_sc(x, src, dst, pad_e):
    T = pad_e // (NW * CHUNK)
    assert T % 2 == 0
    mesh = plsc.VectorSubcoreMesh(core_axis_name="c", subcore_axis_name="s")

    @functools.partial(
        pl.kernel,
        mesh=mesh,
        out_type=jax.ShapeDtypeStruct((pad_e, D), jnp.float32),
        scratch_types=[
            pltpu.VMEM((2, CHUNK), jnp.int32),
            pltpu.VMEM((2, CHUNK), jnp.int32),
            pltpu.VMEM((CHUNK, D), jnp.float32),
            pltpu.VMEM((CHUNK, D), jnp.float32),
            pltpu.VMEM((CHUNK, D), jnp.float32),
            pltpu.VMEM((CHUNK, D), jnp.float32),
            pltpu.SemaphoreType.DMA,
            pltpu.SemaphoreType.DMA,
            pltpu.SemaphoreType.DMA,
            pltpu.SemaphoreType.DMA,
            pltpu.SemaphoreType.DMA,
            pltpu.SemaphoreType.DMA,
        ],
    )
    def sc_kernel(x_hbm, src_hbm, dst_hbm, out_hbm,
                  idx_s, idx_d, rs0, rs1, rd0, rd1,
                  si0, si1, sg0, sg1, sw0, sw1):
        wid = lax.axis_index("s") * 2 + lax.axis_index("c")
        rows_s = (rs0, rs1)
        rows_d = (rd0, rd1)
        sem_i = (si0, si1)
        sem_g = (sg0, sg1)
        sem_w = (sw0, sw1)

        def base_of(t):
            return pl.multiple_of((wid * T + jnp.minimum(t, T - 1)) * CHUNK,
                                  CHUNK)

        def start_idx(t, b):
            base = base_of(t)
            pltpu.async_copy(src_hbm.at[pl.ds(base, CHUNK)], idx_s.at[b],
                             sem_i[b])
            pltpu.async_copy(dst_hbm.at[pl.ds(base, CHUNK)], idx_d.at[b],
                             sem_i[b])

        def wait_idx(b):
            pltpu.make_async_copy(src_hbm.at[pl.ds(0, CHUNK)], idx_s.at[b],
                                  sem_i[b]).wait()
            pltpu.make_async_copy(dst_hbm.at[pl.ds(0, CHUNK)], idx_d.at[b],
                                  sem_i[b]).wait()

        def start_gather(b):
            pltpu.async_copy(x_hbm.at[idx_s.at[b]], rows_s[b], sem_g[b])
            pltpu.async_copy(x_hbm.at[idx_d.at[b]], rows_d[b], sem_g[b])

        def wait_gather(b):
            pltpu.make_async_copy(x_hbm.at[idx_s.at[b]], rows_s[b],
                                  sem_g[b]).wait()
            pltpu.make_async_copy(x_hbm.at[idx_d.at[b]], rows_d[b],
                                  sem_g[b]).wait()

        def start_write(t, b):
            pltpu.async_copy(rows_s[b], out_hbm.at[pl.ds(base_of(t), CHUNK)],
                             sem_w[b])

        def wait_write(b):
            pltpu.make_async_copy(rows_s[b], out_hbm.at[pl.ds(0, CHUNK)],
                                  sem_w[b]).wait()

        start_idx(0, 0)
        start_idx(1, 1)
        wait_idx(0)
        start_gather(0)

        def half_body(t, b):
            nb = 1 - b
            wait_idx(nb)

            @pl.when(t > 0)
            def _():
                wait_write(nb)

            start_gather(nb)
            wait_gather(b)
            start_idx(t + 2, b)

            def row_body(r, c):
                for k in range(D // 16):
                    sl = pl.ds(k * 16, 16)
                    rows_s[b][r, sl] = jnp.abs(rows_s[b][r, sl]
                                               - rows_d[b][r, sl])
                return c

            lax.fori_loop(0, CHUNK, row_body, 0, unroll=2)
            start_write(t, b)

        def pair_body(i, carry):
            half_body(2 * i, 0)
            half_body(2 * i + 1, 1)
            return carry

        lax.fori_loop(0, T // 2, pair_body, 0)

        wait_idx(1)
        wait_gather(0)
        wait_write(1)

    return sc_kernel(x, src, dst)


def _mlp_tc(dif, W1, b1, W2, b2):
    pad_e = dif.shape[0]
    be = 2048
    assert pad_e % be == 0

    def body(d_ref, w1_ref, b1_ref, w2_ref, b2_ref, o_ref):
        h = jnp.dot(d_ref[...], w1_ref[...], preferred_element_type=jnp.float32)
        h = jnp.maximum(h + b1_ref[...], 0.0)
        o_ref[...] = (
            jnp.dot(h, w2_ref[...], preferred_element_type=jnp.float32)
            + b2_ref[...]
        )

    return pl.pallas_call(
        body,
        grid=(pad_e // be,),
        in_specs=[
            pl.BlockSpec((be, D), lambda i: (i, 0)),
            pl.BlockSpec((D, H), lambda i: (0, 0)),
            pl.BlockSpec((1, H), lambda i: (0, 0)),
            pl.BlockSpec((H, 2), lambda i: (0, 0)),
            pl.BlockSpec((1, 2), lambda i: (0, 0)),
        ],
        out_specs=pl.BlockSpec((be, 2), lambda i: (i, 0)),
        out_shape=jax.ShapeDtypeStruct((pad_e, 2), jnp.float32),
    )(dif, W1, b1.reshape(1, H), W2, b2.reshape(1, 2))


def kernel(x, edge_index, W1, b1, W2, b2):
    e = edge_index.shape[1]
    unit = NW * CHUNK * 2
    pad_e = -(-e // unit) * unit
    src = jnp.pad(edge_index[0], (0, pad_e - e))
    dst = jnp.pad(edge_index[1], (0, pad_e - e))
    dif = _absdiff_sc(x, src, dst, pad_e)
    return _mlp_tc(dif, W1, b1, W2, b2)[:e]


# speedup vs baseline: 1.7667x; 1.7667x over previous
import functools

import jax
import jax.numpy as jnp
from jax import lax
from jax.experimental import pallas as pl
from jax.experimental.pallas import tpu as pltpu
from jax.experimental.pallas import tpu_sc as plsc

D = 128
H = 64
CHUNK = 64
NW = 32
NS = 16


def _absdiff_sc(x, src, dst, pad_e):
    n_nodes = x.shape[0]
    rows_per_tile = (n_nodes // NS) // 8 * 8
    tail = n_nodes - rows_per_tile * NS
    T = pad_e // (NW * CHUNK)
    assert T % 2 == 0
    mesh = plsc.VectorSubcoreMesh(core_axis_name="c", subcore_axis_name="s")

    @functools.partial(
        pl.kernel,
        mesh=mesh,
        out_type=jax.ShapeDtypeStruct((pad_e, D), jnp.float32),
        scratch_types=[
            pltpu.VMEM_SHARED((n_nodes, D), jnp.float32),
            pltpu.VMEM((2, CHUNK), jnp.int32),
            pltpu.VMEM((2, CHUNK), jnp.int32),
            pltpu.VMEM((CHUNK, D), jnp.float32),
            pltpu.VMEM((CHUNK, D), jnp.float32),
            pltpu.VMEM((CHUNK, D), jnp.float32),
            pltpu.VMEM((CHUNK, D), jnp.float32),
            pltpu.SemaphoreType.DMA,
            pltpu.SemaphoreType.DMA,
            pltpu.SemaphoreType.DMA,
            pltpu.SemaphoreType.DMA,
            pltpu.SemaphoreType.DMA,
            pltpu.SemaphoreType.DMA,
        ],
    )
    def sc_kernel(x_hbm, src_hbm, dst_hbm, out_hbm,
                  x_sh, idx_s, idx_d, rs0, rs1, rd0, rd1,
                  si0, si1, sg0, sg1, sw0, sw1):
        sid = lax.axis_index("s")
        wid = sid * 2 + lax.axis_index("c")
        rows_s = (rs0, rs1)
        rows_d = (rd0, rd1)
        sem_i = (si0, si1)
        sem_g = (sg0, sg1)
        sem_w = (sw0, sw1)

        stage = pl.multiple_of(sid * rows_per_tile, 8)
        pltpu.sync_copy(x_hbm.at[pl.ds(stage, rows_per_tile)],
                        x_sh.at[pl.ds(stage, rows_per_tile)])
        if tail:
            @pl.when(sid == 0)
            def _():
                base = rows_per_tile * NS
                pltpu.sync_copy(x_hbm.at[pl.ds(base, tail)],
                                x_sh.at[pl.ds(base, tail)])
        plsc.subcore_barrier()

        def base_of(t):
            return pl.multiple_of((wid * T + jnp.minimum(t, T - 1)) * CHUNK,
                                  CHUNK)

        def start_idx(t, b):
            base = base_of(t)
            pltpu.async_copy(src_hbm.at[pl.ds(base, CHUNK)], idx_s.at[b],
                             sem_i[b])
            pltpu.async_copy(dst_hbm.at[pl.ds(base, CHUNK)], idx_d.at[b],
                             sem_i[b])

        def wait_idx(b):
            pltpu.make_async_copy(src_hbm.at[pl.ds(0, CHUNK)], idx_s.at[b],
                                  sem_i[b]).wait()
            pltpu.make_async_copy(dst_hbm.at[pl.ds(0, CHUNK)], idx_d.at[b],
                                  sem_i[b]).wait()

        def start_gather(b):
            pltpu.async_copy(x_sh.at[idx_s.at[b]], rows_s[b], sem_g[b])
            pltpu.async_copy(x_sh.at[idx_d.at[b]], rows_d[b], sem_g[b])

        def wait_gather(b):
            pltpu.make_async_copy(x_sh.at[idx_s.at[b]], rows_s[b],
                                  sem_g[b]).wait()
            pltpu.make_async_copy(x_sh.at[idx_d.at[b]], rows_d[b],
                                  sem_g[b]).wait()

        def start_write(t, b):
            pltpu.async_copy(rows_s[b], out_hbm.at[pl.ds(base_of(t), CHUNK)],
                             sem_w[b])

        def wait_write(b):
            pltpu.make_async_copy(rows_s[b], out_hbm.at[pl.ds(0, CHUNK)],
                                  sem_w[b]).wait()

        start_idx(0, 0)
        start_idx(1, 1)
        wait_idx(0)
        start_gather(0)

        def half_body(t, b):
            nb = 1 - b
            wait_idx(nb)

            @pl.when(t > 0)
            def _():
                wait_write(nb)

            start_gather(nb)
            wait_gather(b)
            start_idx(t + 2, b)

            def row_body(r, c):
                for k in range(D // 16):
                    sl = pl.ds(k * 16, 16)
                    rows_s[b][r, sl] = jnp.abs(rows_s[b][r, sl]
                                               - rows_d[b][r, sl])
                return c

            lax.fori_loop(0, CHUNK, row_body, 0, unroll=2)
            start_write(t, b)

        def pair_body(i, carry):
            half_body(2 * i, 0)
            half_body(2 * i + 1, 1)
            return carry

        lax.fori_loop(0, T // 2, pair_body, 0)

        wait_idx(1)
        wait_gather(0)
        wait_write(1)

    return sc_kernel(x, src, dst)


def _mlp_tc(dif, W1, b1, W2, b2, e):
    be = 2000
    assert e % be == 0

    def body(d_ref, w1_ref, b1_ref, w2_ref, b2_ref, o_ref):
        h = jnp.dot(d_ref[...], w1_ref[...], preferred_element_type=jnp.float32)
        h = jnp.maximum(h + b1_ref[...], 0.0)
        o_ref[...] = (
            jnp.dot(h, w2_ref[...], preferred_element_type=jnp.float32)
            + b2_ref[...]
        )

    return pl.pallas_call(
        body,
        grid=(e // be,),
        in_specs=[
            pl.BlockSpec((be, D), lambda i: (i, 0)),
            pl.BlockSpec((D, H), lambda i: (0, 0)),
            pl.BlockSpec((1, H), lambda i: (0, 0)),
            pl.BlockSpec((H, 2), lambda i: (0, 0)),
            pl.BlockSpec((1, 2), lambda i: (0, 0)),
        ],
        out_specs=pl.BlockSpec((be, 2), lambda i: (i, 0)),
        out_shape=jax.ShapeDtypeStruct((e, 2), jnp.float32),
    )(dif, W1, b1.reshape(1, H), W2, b2.reshape(1, 2))


def kernel(x, edge_index, W1, b1, W2, b2):
    e = edge_index.shape[1]
    unit = NW * CHUNK * 2
    pad_e = -(-e // unit) * unit
    src = jnp.pad(edge_index[0], (0, pad_e - e))
    dst = jnp.pad(edge_index[1], (0, pad_e - e))
    dif = _absdiff_sc(x, src, dst, pad_e)
    return _mlp_tc(dif, W1, b1, W2, b2, e)


# speedup vs baseline: 1.9581x; 1.1083x over previous
import functools

import jax
import jax.numpy as jnp
from jax import lax
from jax.experimental import pallas as pl
from jax.experimental.pallas import tpu as pltpu
from jax.experimental.pallas import tpu_sc as plsc

D = 128
H = 64
CHUNK = 64
NW = 32
NS = 16


def _absdiff_sc(x, src, dst, pad_e):
    n_nodes = x.shape[0]
    rows_per_tile = (n_nodes // NS) // 8 * 8
    tail = n_nodes - rows_per_tile * NS
    T = pad_e // (NW * CHUNK)
    assert T % 2 == 0
    mesh = plsc.VectorSubcoreMesh(core_axis_name="c", subcore_axis_name="s")

    @functools.partial(
        pl.kernel,
        mesh=mesh,
        out_type=jax.ShapeDtypeStruct((pad_e, D), jnp.float32),
        scratch_types=[
            pltpu.VMEM_SHARED((n_nodes, D), jnp.float32),
            pltpu.VMEM((2, CHUNK), jnp.int32),
            pltpu.VMEM((2, CHUNK), jnp.int32),
            pltpu.VMEM((CHUNK, D), jnp.float32),
            pltpu.VMEM((CHUNK, D), jnp.float32),
            pltpu.VMEM((CHUNK, D), jnp.float32),
            pltpu.VMEM((CHUNK, D), jnp.float32),
            pltpu.SemaphoreType.DMA,
            pltpu.SemaphoreType.DMA,
            pltpu.SemaphoreType.DMA,
            pltpu.SemaphoreType.DMA,
            pltpu.SemaphoreType.DMA,
            pltpu.SemaphoreType.DMA,
        ],
    )
    def sc_kernel(x_hbm, src_hbm, dst_hbm, out_hbm,
                  x_sh, idx_s, idx_d, rs0, rs1, rd0, rd1,
                  si0, si1, sg0, sg1, sw0, sw1):
        sid = lax.axis_index("s")
        wid = sid * 2 + lax.axis_index("c")
        rows_s = (rs0, rs1)
        rows_d = (rd0, rd1)
        sem_i = (si0, si1)
        sem_g = (sg0, sg1)
        sem_w = (sw0, sw1)

        stage = pl.multiple_of(sid * rows_per_tile, 8)
        pltpu.sync_copy(x_hbm.at[pl.ds(stage, rows_per_tile)],
                        x_sh.at[pl.ds(stage, rows_per_tile)])
        if tail:
            @pl.when(sid == 0)
            def _():
                base = rows_per_tile * NS
                pltpu.sync_copy(x_hbm.at[pl.ds(base, tail)],
                                x_sh.at[pl.ds(base, tail)])
        plsc.subcore_barrier()

        def base_of(t):
            return pl.multiple_of((wid * T + jnp.minimum(t, T - 1)) * CHUNK,
                                  CHUNK)

        def start_idx(t, b):
            base = base_of(t)
            pltpu.async_copy(src_hbm.at[pl.ds(base, CHUNK)], idx_s.at[b],
                             sem_i[b])
            pltpu.async_copy(dst_hbm.at[pl.ds(base, CHUNK)], idx_d.at[b],
                             sem_i[b])

        def wait_idx(b):
            pltpu.make_async_copy(src_hbm.at[pl.ds(0, CHUNK)], idx_s.at[b],
                                  sem_i[b]).wait()
            pltpu.make_async_copy(dst_hbm.at[pl.ds(0, CHUNK)], idx_d.at[b],
                                  sem_i[b]).wait()

        def start_gather(b):
            pltpu.async_copy(x_sh.at[idx_s.at[b]], rows_s[b], sem_g[b])
            pltpu.async_copy(x_sh.at[idx_d.at[b]], rows_d[b], sem_g[b])

        def wait_gather(b):
            pltpu.make_async_copy(x_sh.at[idx_s.at[b]], rows_s[b],
                                  sem_g[b]).wait()
            pltpu.make_async_copy(x_sh.at[idx_d.at[b]], rows_d[b],
                                  sem_g[b]).wait()

        def start_write(t, b):
            pltpu.async_copy(rows_s[b], out_hbm.at[pl.ds(base_of(t), CHUNK)],
                             sem_w[b])

        def wait_write(b):
            pltpu.make_async_copy(rows_s[b], out_hbm.at[pl.ds(0, CHUNK)],
                                  sem_w[b]).wait()

        start_idx(0, 0)
        start_idx(1, 1)
        wait_idx(0)
        start_gather(0)

        def half_body(t, b):
            nb = 1 - b
            wait_idx(nb)

            @pl.when(t > 0)
            def _():
                wait_write(nb)

            start_gather(nb)
            wait_gather(b)
            start_idx(t + 2, b)

            def row_body(i, c):
                for k in range(D // 16):
                    sl = (i, pl.ds(k * 16, 16))
                    rows_s[b][sl] = jnp.abs(rows_s[b][sl] - rows_d[b][sl])
                return c

            lax.fori_loop(0, CHUNK, row_body, 0, unroll=2)
            start_write(t, b)

        def pair_body(i, carry):
            half_body(2 * i, 0)
            half_body(2 * i + 1, 1)
            return carry

        lax.fori_loop(0, T // 2, pair_body, 0)

        wait_idx(1)
        wait_gather(0)
        wait_write(1)

    return sc_kernel(x, src, dst)


def _mlp_tc(dif, W1, b1, W2, b2, e):
    be = 8000
    assert e % be == 0

    def body(d_ref, w1_ref, b1_ref, w2_ref, b2_ref, o_ref):
        h = jnp.dot(d_ref[...], w1_ref[...], preferred_element_type=jnp.float32)
        h = jnp.maximum(h + b1_ref[...], 0.0)
        o_ref[...] = (
            jnp.dot(h, w2_ref[...], preferred_element_type=jnp.float32)
            + b2_ref[...]
        )

    return pl.pallas_call(
        body,
        grid=(e // be,),
        in_specs=[
            pl.BlockSpec((be, D), lambda i: (i, 0)),
            pl.BlockSpec((D, H), lambda i: (0, 0)),
            pl.BlockSpec((1, H), lambda i: (0, 0)),
            pl.BlockSpec((H, 2), lambda i: (0, 0)),
            pl.BlockSpec((1, 2), lambda i: (0, 0)),
        ],
        out_specs=pl.BlockSpec((be, 2), lambda i: (i, 0)),
        out_shape=jax.ShapeDtypeStruct((e, 2), jnp.float32),
    )(dif, W1, b1.reshape(1, H), W2, b2.reshape(1, 2))


def kernel(x, edge_index, W1, b1, W2, b2):
    e = edge_index.shape[1]
    unit = NW * CHUNK * 2
    pad_e = -(-e // unit) * unit
    src = jnp.pad(edge_index[0], (0, pad_e - e))
    dst = jnp.pad(edge_index[1], (0, pad_e - e))
    dif = _absdiff_sc(x, src, dst, pad_e)
    return _mlp_tc(dif, W1, b1, W2, b2, e)


# speedup vs baseline: 2.1114x; 1.0783x over previous
import functools

import jax
import jax.numpy as jnp
from jax import lax
from jax.experimental import pallas as pl
from jax.experimental.pallas import tpu as pltpu
from jax.experimental.pallas import tpu_sc as plsc

D = 128
H = 64
CHUNK = 64
NW = 32
NS = 16


def _absdiff_sc(x, src, dst, pad_e):
    n_nodes = x.shape[0]
    rows_per_tile = (n_nodes // NS) // 8 * 8
    tail = n_nodes - rows_per_tile * NS
    T = pad_e // (NW * CHUNK)
    assert T % 2 == 0
    mesh = plsc.VectorSubcoreMesh(core_axis_name="c", subcore_axis_name="s")

    @functools.partial(
        pl.kernel,
        mesh=mesh,
        out_type=jax.ShapeDtypeStruct((pad_e, D), jnp.float32),
        scratch_types=[
            pltpu.VMEM_SHARED((n_nodes, D), jnp.float32),
            pltpu.VMEM((2, CHUNK), jnp.int32),
            pltpu.VMEM((2, CHUNK), jnp.int32),
            pltpu.VMEM((CHUNK, D), jnp.float32),
            pltpu.VMEM((CHUNK, D), jnp.float32),
            pltpu.VMEM((CHUNK, D), jnp.float32),
            pltpu.VMEM((CHUNK, D), jnp.float32),
            pltpu.SemaphoreType.DMA,
            pltpu.SemaphoreType.DMA,
            pltpu.SemaphoreType.DMA,
            pltpu.SemaphoreType.DMA,
            pltpu.SemaphoreType.DMA,
            pltpu.SemaphoreType.DMA,
        ],
    )
    def sc_kernel(x_hbm, src_hbm, dst_hbm, out_hbm,
                  x_sh, idx_s, idx_d, rs0, rs1, rd0, rd1,
                  si0, si1, sg0, sg1, sw0, sw1):
        sid = lax.axis_index("s")
        wid = sid * 2 + lax.axis_index("c")
        rows_s = (rs0, rs1)
        rows_d = (rd0, rd1)
        sem_i = (si0, si1)
        sem_g = (sg0, sg1)
        sem_w = (sw0, sw1)

        stage = pl.multiple_of(sid * rows_per_tile, 8)
        pltpu.sync_copy(x_hbm.at[pl.ds(stage, rows_per_tile)],
                        x_sh.at[pl.ds(stage, rows_per_tile)])
        if tail:
            @pl.when(sid == 0)
            def _():
                base = rows_per_tile * NS
                pltpu.sync_copy(x_hbm.at[pl.ds(base, tail)],
                                x_sh.at[pl.ds(base, tail)])
        plsc.subcore_barrier()

        def base_of(t):
            return pl.multiple_of((wid * T + jnp.minimum(t, T - 1)) * CHUNK,
                                  CHUNK)

        def start_idx(t, b):
            base = base_of(t)
            pltpu.async_copy(src_hbm.at[pl.ds(base, CHUNK)], idx_s.at[b],
                             sem_i[b])
            pltpu.async_copy(dst_hbm.at[pl.ds(base, CHUNK)], idx_d.at[b],
                             sem_i[b])

        def wait_idx(b):
            pltpu.make_async_copy(src_hbm.at[pl.ds(0, CHUNK)], idx_s.at[b],
                                  sem_i[b]).wait()
            pltpu.make_async_copy(dst_hbm.at[pl.ds(0, CHUNK)], idx_d.at[b],
                                  sem_i[b]).wait()

        def start_gather(b):
            pltpu.async_copy(x_sh.at[idx_s.at[b]], rows_s[b], sem_g[b])
            pltpu.async_copy(x_sh.at[idx_d.at[b]], rows_d[b], sem_g[b])

        def wait_gather(b):
            pltpu.make_async_copy(x_sh.at[idx_s.at[b]], rows_s[b],
                                  sem_g[b]).wait()
            pltpu.make_async_copy(x_sh.at[idx_d.at[b]], rows_d[b],
                                  sem_g[b]).wait()

        def start_write(t, b):
            pltpu.async_copy(rows_s[b], out_hbm.at[pl.ds(base_of(t), CHUNK)],
                             sem_w[b])

        def wait_write(b):
            pltpu.make_async_copy(rows_s[b], out_hbm.at[pl.ds(0, CHUNK)],
                                  sem_w[b]).wait()

        start_idx(0, 0)
        start_idx(1, 1)
        wait_idx(0)
        start_gather(0)

        def half_body(t, b):
            nb = 1 - b
            wait_idx(nb)

            @pl.when(t > 0)
            def _():
                wait_write(nb)

            start_gather(nb)
            wait_gather(b)
            start_idx(t + 2, b)

            def row_body(i, c):
                for k in range(D // 16):
                    sl = (i, pl.ds(k * 16, 16))
                    rows_s[b][sl] = jnp.abs(rows_s[b][sl] - rows_d[b][sl])
                return c

            lax.fori_loop(0, CHUNK, row_body, 0, unroll=2)
            start_write(t, b)

        def pair_body(i, carry):
            half_body(2 * i, 0)
            half_body(2 * i + 1, 1)
            return carry

        lax.fori_loop(0, T // 2, pair_body, 0)

        wait_idx(1)
        wait_gather(0)
        wait_write(1)

    return sc_kernel(x, src, dst)


def _mlp_tc(dif, W1, b1, W2, b2, e):
    be = 8000
    assert e % be == 0

    def body(d_ref, w1_ref, b1_ref, w2_ref, b2_ref, o_ref):
        h = jnp.dot(d_ref[...], w1_ref[...], preferred_element_type=jnp.float32)
        h = jnp.maximum(h + b1_ref[...], 0.0)
        o_ref[...] = (
            jnp.dot(h, w2_ref[...], preferred_element_type=jnp.float32)
            + b2_ref[...]
        )

    return pl.pallas_call(
        body,
        grid=(e // be,),
        in_specs=[
            pl.BlockSpec((be, D), lambda i: (i, 0)),
            pl.BlockSpec((D, H), lambda i: (0, 0)),
            pl.BlockSpec((1, H), lambda i: (0, 0)),
            pl.BlockSpec((H, 2), lambda i: (0, 0)),
            pl.BlockSpec((1, 2), lambda i: (0, 0)),
        ],
        out_specs=pl.BlockSpec((be, 2), lambda i: (i, 0)),
        out_shape=jax.ShapeDtypeStruct((e, 2), jnp.float32),
    )(dif, W1, b1.reshape(1, H), W2, b2.reshape(1, 2))


def kernel(x, edge_index, W1, b1, W2, b2):
    e = edge_index.shape[1]
    S = 4
    assert e % S == 0
    es = e // S
    unit = NW * CHUNK * 2
    pad_s = -(-es // unit) * unit
    outs = []
    for i in range(S):
        src = jnp.pad(lax.slice_in_dim(edge_index[0], i * es, (i + 1) * es),
                      (0, pad_s - es))
        dst = jnp.pad(lax.slice_in_dim(edge_index[1], i * es, (i + 1) * es),
                      (0, pad_s - es))
        dif = _absdiff_sc(x, src, dst, pad_s)
        outs.append(_mlp_tc(dif, W1, b1, W2, b2, es))
    return jnp.concatenate(outs, axis=0)


# speedup vs baseline: 2.1120x; 1.0003x over previous
import functools

import jax
import jax.numpy as jnp
from jax import lax
from jax.experimental import pallas as pl
from jax.experimental.pallas import tpu as pltpu
from jax.experimental.pallas import tpu_sc as plsc

D = 128
H = 64
CHUNK = 64
NW = 32
NS = 16


def _absdiff_sc(x, src, dst, pad_e):
    n_nodes = x.shape[0]
    rows_per_tile = (n_nodes // NS) // 8 * 8
    tail = n_nodes - rows_per_tile * NS
    T = pad_e // (NW * CHUNK)
    assert T % 2 == 0
    mesh = plsc.VectorSubcoreMesh(core_axis_name="c", subcore_axis_name="s")

    @functools.partial(
        pl.kernel,
        mesh=mesh,
        out_type=jax.ShapeDtypeStruct((pad_e, D), jnp.float32),
        scratch_types=[
            pltpu.VMEM_SHARED((n_nodes, D), jnp.float32),
            pltpu.VMEM((2, CHUNK), jnp.int32),
            pltpu.VMEM((2, CHUNK), jnp.int32),
            pltpu.VMEM((CHUNK, D), jnp.float32),
            pltpu.VMEM((CHUNK, D), jnp.float32),
            pltpu.VMEM((CHUNK, D), jnp.float32),
            pltpu.VMEM((CHUNK, D), jnp.float32),
            pltpu.SemaphoreType.DMA,
            pltpu.SemaphoreType.DMA,
            pltpu.SemaphoreType.DMA,
            pltpu.SemaphoreType.DMA,
            pltpu.SemaphoreType.DMA,
            pltpu.SemaphoreType.DMA,
        ],
    )
    def sc_kernel(x_hbm, src_hbm, dst_hbm, out_hbm,
                  x_sh, idx_s, idx_d, rs0, rs1, rd0, rd1,
                  si0, si1, sg0, sg1, sw0, sw1):
        sid = lax.axis_index("s")
        wid = sid * 2 + lax.axis_index("c")
        rows_s = (rs0, rs1)
        rows_d = (rd0, rd1)
        sem_i = (si0, si1)
        sem_g = (sg0, sg1)
        sem_w = (sw0, sw1)

        stage = pl.multiple_of(sid * rows_per_tile, 8)
        pltpu.sync_copy(x_hbm.at[pl.ds(stage, rows_per_tile)],
                        x_sh.at[pl.ds(stage, rows_per_tile)])
        if tail:
            @pl.when(sid == 0)
            def _():
                base = rows_per_tile * NS
                pltpu.sync_copy(x_hbm.at[pl.ds(base, tail)],
                                x_sh.at[pl.ds(base, tail)])
        plsc.subcore_barrier()

        def base_of(t):
            return pl.multiple_of((wid * T + jnp.minimum(t, T - 1)) * CHUNK,
                                  CHUNK)

        def start_idx(t, b):
            base = base_of(t)
            pltpu.async_copy(src_hbm.at[pl.ds(base, CHUNK)], idx_s.at[b],
                             sem_i[b])
            pltpu.async_copy(dst_hbm.at[pl.ds(base, CHUNK)], idx_d.at[b],
                             sem_i[b])

        def wait_idx(b):
            pltpu.make_async_copy(src_hbm.at[pl.ds(0, CHUNK)], idx_s.at[b],
                                  sem_i[b]).wait()
            pltpu.make_async_copy(dst_hbm.at[pl.ds(0, CHUNK)], idx_d.at[b],
                                  sem_i[b]).wait()

        def start_gather(b):
            pltpu.async_copy(x_sh.at[idx_s.at[b]], rows_s[b], sem_g[b])
            pltpu.async_copy(x_sh.at[idx_d.at[b]], rows_d[b], sem_g[b])

        def wait_gather(b):
            pltpu.make_async_copy(x_sh.at[idx_s.at[b]], rows_s[b],
                                  sem_g[b]).wait()
            pltpu.make_async_copy(x_sh.at[idx_d.at[b]], rows_d[b],
                                  sem_g[b]).wait()

        def start_write(t, b):
            pltpu.async_copy(rows_s[b], out_hbm.at[pl.ds(base_of(t), CHUNK)],
                             sem_w[b])

        def wait_write(b):
            pltpu.make_async_copy(rows_s[b], out_hbm.at[pl.ds(0, CHUNK)],
                                  sem_w[b]).wait()

        start_idx(0, 0)
        start_idx(1, 1)
        wait_idx(0)
        start_gather(0)

        def half_body(t, b):
            nb = 1 - b
            wait_idx(nb)

            @pl.when(t > 0)
            def _():
                wait_write(nb)

            start_gather(nb)
            wait_gather(b)
            start_idx(t + 2, b)

            def row_body(i, c):
                for k in range(D // 16):
                    sl = (i, pl.ds(k * 16, 16))
                    rows_s[b][sl] = jnp.abs(rows_s[b][sl] - rows_d[b][sl])
                return c

            lax.fori_loop(0, CHUNK, row_body, 0, unroll=4)
            start_write(t, b)

        def pair_body(i, carry):
            half_body(2 * i, 0)
            half_body(2 * i + 1, 1)
            return carry

        lax.fori_loop(0, T // 2, pair_body, 0)

        wait_idx(1)
        wait_gather(0)
        wait_write(1)

    return sc_kernel(x, src, dst)


def _mlp_tc(dif, W1, b1, W2, b2, e):
    be = 8000
    assert e % be == 0

    def body(d_ref, w1_ref, b1_ref, w2_ref, b2_ref, o_ref):
        h = jnp.dot(d_ref[...], w1_ref[...], preferred_element_type=jnp.float32)
        h = jnp.maximum(h + b1_ref[...], 0.0)
        o_ref[...] = (
            jnp.dot(h, w2_ref[...], preferred_element_type=jnp.float32)
            + b2_ref[...]
        )

    return pl.pallas_call(
        body,
        grid=(e // be,),
        in_specs=[
            pl.BlockSpec((be, D), lambda i: (i, 0)),
            pl.BlockSpec((D, H), lambda i: (0, 0)),
            pl.BlockSpec((1, H), lambda i: (0, 0)),
            pl.BlockSpec((H, 2), lambda i: (0, 0)),
            pl.BlockSpec((1, 2), lambda i: (0, 0)),
        ],
        out_specs=pl.BlockSpec((be, 2), lambda i: (i, 0)),
        out_shape=jax.ShapeDtypeStruct((e, 2), jnp.float32),
    )(dif, W1, b1.reshape(1, H), W2, b2.reshape(1, 2))


def kernel(x, edge_index, W1, b1, W2, b2):
    e = edge_index.shape[1]
    S = 4
    assert e % S == 0
    es = e // S
    unit = NW * CHUNK * 2
    pad_s = -(-es // unit) * unit
    outs = []
    for i in range(S):
        src = jnp.pad(lax.slice_in_dim(edge_index[0], i * es, (i + 1) * es),
                      (0, pad_s - es))
        dst = jnp.pad(lax.slice_in_dim(edge_index[1], i * es, (i + 1) * es),
                      (0, pad_s - es))
        dif = _absdiff_sc(x, src, dst, pad_s)
        outs.append(_mlp_tc(dif, W1, b1, W2, b2, es))
    return jnp.concatenate(outs, axis=0)
